# baseline (speedup 1.0000x reference)
import jax
import jax.numpy as jnp
from jax import lax
from jax.experimental import pallas as pl
from jax.experimental.pallas import tpu as pltpu
from jax.experimental.pallas import tpu_sc as plsc

T = 262144
VOLUME_SIZE = 128
NVOX = VOLUME_SIZE ** 3
BATCH = 8


TBLK = 32768
NTBLK = T // TBLK


def _rne_bf16_bits(u):
    return (u + 0x7FFF + ((u >> 16) & 1)) & jnp.uint32(0xFFFF0000)


def _values_body(x_ref, w0_ref, b0_ref, w1_ref, b1_ref, w2_ref, b2_ref,
                 w3_ref, b3_ref, w4_ref, b4_ref, w5_ref, b5_ref, rv_ref,
                 i0_ref, i1_ref, i2_ref, valp_ref, flat_ref):
    h = jnp.sin(30.0 * (jnp.dot(x_ref[...], w0_ref[...],
                                preferred_element_type=jnp.float32)
                        + b0_ref[...]))
    for w_ref, b_ref in ((w1_ref, b1_ref), (w2_ref, b2_ref),
                         (w3_ref, b3_ref), (w4_ref, b4_ref)):
        h = h + jnp.sin(jnp.dot(h, w_ref[...],
                                preferred_element_type=jnp.float32)
                        + b_ref[...])
    out = jnp.dot(h, w5_ref[...], preferred_element_type=jnp.float32)
    out = out + b5_ref[0] + rv_ref[0]
    bits = _rne_bf16_bits(lax.bitcast_convert_type(out, jnp.uint32))
    rows = []
    for k in range(4):
        pair = (bits[2 * k:2 * k + 1] >> 16) | bits[2 * k + 1:2 * k + 2]
        rows.append(lax.bitcast_convert_type(pair, jnp.int32).reshape(TBLK))
    valp_ref[...] = jnp.concatenate(rows)
    flat_ref[...] = (i0_ref[...] * (VOLUME_SIZE * VOLUME_SIZE)
                     + i1_ref[...] * VOLUME_SIZE
                     + i2_ref[...]).reshape(TBLK)


def _values_and_flat(x, inds, reference_values, W0, b0, W1, b1, W2, b2,
                     W3, b3, W4, b4, W5, b5):
    i0 = inds[:, 0].reshape(NTBLK, 1, TBLK)
    i1 = inds[:, 1].reshape(NTBLK, 1, TBLK)
    i2 = inds[:, 2].reshape(NTBLK, 1, TBLK)
    b5r = b5.reshape(NTBLK, 1, TBLK)
    rvr = reference_values.reshape(NTBLK, 1, TBLK)

    def full(shape):
        return pl.BlockSpec(shape, lambda i: (0,) * len(shape))

    tb3 = pl.BlockSpec((1, 1, TBLK), lambda i: (i, 0, 0))
    valp, flat = pl.pallas_call(
        _values_body,
        grid=(NTBLK,),
        in_specs=[
            full((BATCH, 64)), full((64, 8)), full((1, 8)),
            full((8, 8)), full((1, 8)), full((8, 8)), full((1, 8)),
            full((8, 8)), full((1, 8)), full((8, 8)), full((1, 8)),
            pl.BlockSpec((BATCH, TBLK), lambda i: (0, i)),
            tb3, tb3,
            tb3, tb3, tb3,
        ],
        out_specs=[
            pl.BlockSpec((TBLK * 4,), lambda i: (i,)),
            pl.BlockSpec((TBLK,), lambda i: (i,)),
        ],
        out_shape=[
            jax.ShapeDtypeStruct((T * 4,), jnp.int32),
            jax.ShapeDtypeStruct((T,), jnp.int32),
        ],
    )(x, W0, b0.reshape(1, -1), W1, b1.reshape(1, -1), W2, b2.reshape(1, -1),
      W3, b3.reshape(1, -1), W4, b4.reshape(1, -1), W5, b5r, rvr, i0, i1, i2)
    return valp, flat


NC = 2
NS = 16
L = 16
NPASS = 16
RANGE = NVOX // (NPASS * NC)
RSHIFT = 16
SLICE = RANGE // NS
ZB = 1024
TPT = T // NS
NGRP = TPT // L
BLK = 1024
NCH = BLK // 128
NQ = BLK // L
SEG = 4096
NSEG = TPT // SEG
SGRP = SEG // L
CAP = SEG + 128


def _scatter_body(flat_hbm, vp_hbm, out_hbm,
                  idx_res, vres, pk, lvb, pays, zbuf,
                  acc, ssem, zsem, rsem):
    c = lax.axis_index("c")
    s = lax.axis_index("s")
    tb = s * TPT
    iota = lax.iota(jnp.int32, L)
    zeros_i = jnp.zeros((L,), jnp.int32)
    zeros_f = jnp.zeros((L,), jnp.float32)

    pltpu.sync_copy(flat_hbm.at[pl.ds(tb, TPT)], idx_res)
    vpb = (s // 2) * (TBLK * 4) + (s % 2) * TPT
    for k in range(4):
        pltpu.sync_copy(vp_hbm.at[pl.ds(vpb + k * TBLK, TPT)],
                        vres.at[pl.ds(k * TPT, TPT)])

    def zinit(j, _):
        zbuf[pl.ds(j * L, L)] = zeros_f
        return _
    lax.fori_loop(0, ZB // L, zinit, None, unroll=8)

    def pinit(j, _):
        for b in range(BATCH):
            pays[b][pl.ds(j * L, L)] = zeros_f
        return _
    lax.fori_loop(0, NQ, pinit, None, unroll=4)

    def cinit(j, _):
        pk[pl.ds(j * L, L)] = zeros_i
        return _
    lax.fori_loop(0, CAP // L, cinit, None, unroll=4)

    def linit(j, _):
        off = j * L + iota
        plsc.store_scatter(lvb, [off >> 7, zeros_i, off & 127], iota * 8)
        return _
    lax.fori_loop(0, BLK // L, linit, None, unroll=4)

    def pass_body(p, hw):
        unit = p * NC + c
        base = unit * RANGE
        plsc.subcore_barrier()
        for b in range(BATCH):
            for h in range(SLICE // ZB):
                pltpu.async_copy(
                    zbuf, acc[b].at[pl.ds(s * SLICE + h * ZB, ZB)], zsem)
        for b in range(BATCH):
            for h in range(SLICE // ZB):
                pltpu.make_async_copy(
                    zbuf, acc[b].at[pl.ds(s * SLICE + h * ZB, ZB)],
                    zsem).wait()
        plsc.subcore_barrier()

        unit_v = jnp.full((L,), unit, jnp.int32)

        def seg_body(seg, hw):
            def scan_body(g, carry):
                cnt_v, gpos = carry
                fl = idx_res[pl.ds(seg * SEG + g * L, L)]
                m = (fl >> RSHIFT) == unit_v
                packed = (gpos << 17) | (fl & (RANGE - 1))
                pc = plsc.cumsum(m.astype(jnp.int32))
                off = cnt_v + pc - 1
                plsc.store_scatter(pk, [off], packed, mask=m)
                cnt_v = cnt_v + plsc.all_reduce_population_count(m)
                return cnt_v, gpos + L
            cnt_v, _ = lax.fori_loop(0, SGRP, scan_body,
                                     (zeros_i, seg * SEG + iota), unroll=4)
            cnt = jnp.max(cnt_v)

            nblk = (cnt + BLK - 1) // BLK

            def blk_body(blk, hw):
                cb = jnp.minimum(cnt - blk * BLK, BLK)
                ng = (cb + 127) >> 7
                cb_v = jnp.full((L,), cb, jnp.int32)

                def ext_body(q, _):
                    q16 = iota + q * L
                    sel = q16 < cb_v
                    pkv = pk[pl.ds(blk * BLK + q * L, L)]
                    pos = lax.shift_right_logical(pkv, 17)
                    lv = pkv & (RANGE - 1)
                    plsc.store_scatter(lvb,
                                       [jnp.full((L,), q >> 3, jnp.int32),
                                        zeros_i, (q & 7) * L + iota], lv)
                    for k in range(4):
                        g32 = plsc.load_gather(vres, [pos + k * TPT])
                        bf = plsc.bitcast(g32, jnp.bfloat16)
                        a, b2 = plsc.unpack(
                            bf, format=plsc.PackFormat.INTERLEAVED)
                        pays[2 * k][pl.ds(q * L, L)] = jnp.where(sel, a, 0.0)
                        pays[2 * k + 1][pl.ds(q * L, L)] = jnp.where(
                            sel, b2, 0.0)
                    return _
                nq = (cb + L - 1) >> 4
                lax.fori_loop(0, nq, ext_body, None)

                def zpay(r, _):
                    for b in range(BATCH):
                        pays[b][pl.ds((nq + r) * L, L)] = zeros_f
                    return _
                lax.fori_loop(0, jnp.maximum(hw - nq, 0), zpay, None)
                hw = jnp.maximum(hw, nq)

                def st_body(j, _):
                    for b in range(BATCH):
                        pltpu.async_copy(pays[b].at[pl.ds(j * 128, 128)],
                                         acc[b].at[lvb.at[j, 0]], ssem,
                                         add=True)
                    for b in range(BATCH):
                        pltpu.make_async_copy(
                            pays[b].at[pl.ds(j * 128, 128)],
                            acc[b].at[lvb.at[j, 0]], ssem).wait()
                    return _
                lax.fori_loop(0, ng, st_body, None)
                return hw
            return lax.fori_loop(0, nblk, blk_body, hw)
        hw = lax.fori_loop(0, NSEG, seg_body, hw)

        plsc.subcore_barrier()
        for b in range(BATCH):
            pltpu.async_copy(acc[b].at[pl.ds(s * SLICE, SLICE)],
                             out_hbm.at[b, pl.ds(base + s * SLICE, SLICE)],
                             rsem)
        for b in range(BATCH):
            pltpu.make_async_copy(
                acc[b].at[pl.ds(s * SLICE, SLICE)],
                out_hbm.at[b, pl.ds(base + s * SLICE, SLICE)], rsem).wait()
        return hw

    lax.fori_loop(0, NPASS, pass_body, jnp.int32(0))


def _scatter(flat, values_p):
    mesh = plsc.VectorSubcoreMesh(core_axis_name="c", subcore_axis_name="s",
                                  num_cores=NC, num_subcores=NS)
    return pl.kernel(
        _scatter_body,
        out_type=jax.ShapeDtypeStruct((BATCH, NVOX), jnp.float32),
        mesh=mesh,
        scratch_types=[
            pltpu.VMEM((TPT,), jnp.int32),
            pltpu.VMEM((TPT * 4,), jnp.int32),
            pltpu.VMEM((CAP,), jnp.int32),
            pltpu.VMEM((NCH, 1, 128), jnp.int32),
            [pltpu.VMEM((BLK,), jnp.float32) for _ in range(BATCH)],
            pltpu.VMEM((ZB,), jnp.float32),
            [pltpu.VMEM_SHARED((RANGE,), jnp.float32)
             for _ in range(BATCH)],
            pltpu.SemaphoreType.DMA,
            pltpu.SemaphoreType.DMA,
            pltpu.SemaphoreType.DMA,
        ],
        compiler_params=pltpu.CompilerParams(needs_layout_passes=False),
    )(flat, values_p)


def kernel(x, inds, reference_values, W0, b0, W1, b1, W2, b2, W3, b3, W4, b4,
           W5, b5):
    values_p, flat = _values_and_flat(
        x, inds, reference_values, W0, b0, W1, b1, W2, b2, W3, b3, W4, b4,
        W5, b5)
    grids = _scatter(flat, values_p)
    return grids.reshape(BATCH, VOLUME_SIZE, VOLUME_SIZE, VOLUME_SIZE)


# speedup vs baseline: 1.1240x; 1.1240x over previous
import jax
import jax.numpy as jnp
from jax import lax
from jax.experimental import pallas as pl
from jax.experimental.pallas import tpu as pltpu
from jax.experimental.pallas import tpu_sc as plsc

T = 262144
VOLUME_SIZE = 128
NVOX = VOLUME_SIZE ** 3
BATCH = 8


TBLK = 32768
NTBLK = T // TBLK


def _rne_bf16_bits(u):
    return (u + 0x7FFF + ((u >> 16) & 1)) & jnp.uint32(0xFFFF0000)


def _values_body(x_ref, w0_ref, b0_ref, w1_ref, b1_ref, w2_ref, b2_ref,
                 w3_ref, b3_ref, w4_ref, b4_ref, w5_ref, b5_ref, rv_ref,
                 i0_ref, i1_ref, i2_ref, valp_ref, flat_ref):
    h = jnp.sin(30.0 * (jnp.dot(x_ref[...], w0_ref[...],
                                preferred_element_type=jnp.float32)
                        + b0_ref[...]))
    for w_ref, b_ref in ((w1_ref, b1_ref), (w2_ref, b2_ref),
                         (w3_ref, b3_ref), (w4_ref, b4_ref)):
        h = h + jnp.sin(jnp.dot(h, w_ref[...],
                                preferred_element_type=jnp.float32)
                        + b_ref[...])
    out = jnp.dot(h, w5_ref[...], preferred_element_type=jnp.float32)
    out = out + b5_ref[0] + rv_ref[0]
    bits = _rne_bf16_bits(lax.bitcast_convert_type(out, jnp.uint32))
    rows = []
    for k in range(4):
        pair = (bits[2 * k:2 * k + 1] >> 16) | bits[2 * k + 1:2 * k + 2]
        rows.append(lax.bitcast_convert_type(pair, jnp.int32).reshape(TBLK))
    valp_ref[...] = jnp.concatenate(rows)
    flat_ref[...] = (i0_ref[...] * (VOLUME_SIZE * VOLUME_SIZE)
                     + i1_ref[...] * VOLUME_SIZE
                     + i2_ref[...]).reshape(TBLK)


def _values_and_flat(x, inds, reference_values, W0, b0, W1, b1, W2, b2,
                     W3, b3, W4, b4, W5, b5):
    i0 = inds[:, 0].reshape(NTBLK, 1, TBLK)
    i1 = inds[:, 1].reshape(NTBLK, 1, TBLK)
    i2 = inds[:, 2].reshape(NTBLK, 1, TBLK)
    b5r = b5.reshape(NTBLK, 1, TBLK)
    rvr = reference_values.reshape(NTBLK, 1, TBLK)

    def full(shape):
        return pl.BlockSpec(shape, lambda i: (0,) * len(shape))

    tb3 = pl.BlockSpec((1, 1, TBLK), lambda i: (i, 0, 0))
    valp, flat = pl.pallas_call(
        _values_body,
        grid=(NTBLK,),
        in_specs=[
            full((BATCH, 64)), full((64, 8)), full((1, 8)),
            full((8, 8)), full((1, 8)), full((8, 8)), full((1, 8)),
            full((8, 8)), full((1, 8)), full((8, 8)), full((1, 8)),
            pl.BlockSpec((BATCH, TBLK), lambda i: (0, i)),
            tb3, tb3,
            tb3, tb3, tb3,
        ],
        out_specs=[
            pl.BlockSpec((TBLK * 4,), lambda i: (i,)),
            pl.BlockSpec((TBLK,), lambda i: (i,)),
        ],
        out_shape=[
            jax.ShapeDtypeStruct((T * 4,), jnp.int32),
            jax.ShapeDtypeStruct((T,), jnp.int32),
        ],
    )(x, W0, b0.reshape(1, -1), W1, b1.reshape(1, -1), W2, b2.reshape(1, -1),
      W3, b3.reshape(1, -1), W4, b4.reshape(1, -1), W5, b5r, rvr, i0, i1, i2)
    return valp, flat


NC = 2
NS = 16
L = 16
NPASS = 16
RANGE = NVOX // (NPASS * NC)
RSHIFT = 16
SLICE = RANGE // NS
ZB = 1024
TPT = T // NS
NGRP = TPT // L
BLK = 1024
NCH = BLK // 128
NQ = BLK // L
SEG = 4096
NSEG = TPT // SEG
SGRP = SEG // L
CAP = SEG + 128


def _scatter_body(flat_hbm, vp_hbm, out_hbm,
                  idx_res, vres, pk, lvb, pays, zbuf,
                  acc, ssem, zsem, rsem):
    c = lax.axis_index("c")
    s = lax.axis_index("s")
    tb = s * TPT
    iota = lax.iota(jnp.int32, L)
    zeros_i = jnp.zeros((L,), jnp.int32)
    zeros_f = jnp.zeros((L,), jnp.float32)

    pltpu.sync_copy(flat_hbm.at[pl.ds(tb, TPT)], idx_res)
    vpb = (s // 2) * (TBLK * 4) + (s % 2) * TPT
    for k in range(4):
        pltpu.sync_copy(vp_hbm.at[pl.ds(vpb + k * TBLK, TPT)],
                        vres.at[pl.ds(k * TPT, TPT)])

    def zinit(j, _):
        zbuf[pl.ds(j * L, L)] = zeros_f
        return _
    lax.fori_loop(0, ZB // L, zinit, None, unroll=8)

    def pinit(j, _):
        for b in range(BATCH):
            pays[b][pl.ds(j * L, L)] = zeros_f
        return _
    lax.fori_loop(0, NQ, pinit, None, unroll=4)

    def cinit(j, _):
        pk[pl.ds(j * L, L)] = zeros_i
        return _
    lax.fori_loop(0, CAP // L, cinit, None, unroll=4)

    def linit(j, _):
        off = j * L + iota
        plsc.store_scatter(lvb, [off >> 7, zeros_i, off & 127], iota * 8)
        return _
    lax.fori_loop(0, BLK // L, linit, None, unroll=4)

    def pass_body(p, hw):
        unit = p * NC + c
        base = unit * RANGE
        plsc.subcore_barrier()
        for b in range(BATCH):
            for h in range(SLICE // ZB):
                pltpu.async_copy(
                    zbuf, acc[b].at[pl.ds(s * SLICE + h * ZB, ZB)], zsem)
        for b in range(BATCH):
            for h in range(SLICE // ZB):
                pltpu.make_async_copy(
                    zbuf, acc[b].at[pl.ds(s * SLICE + h * ZB, ZB)],
                    zsem).wait()
        plsc.subcore_barrier()

        unit_v = jnp.full((L,), unit, jnp.int32)

        def seg_body(seg, hw):
            def scan_body(g, carry):
                cnt_v, gpos = carry
                fl = idx_res[pl.ds(seg * SEG + g * L, L)]
                m = (fl >> RSHIFT) == unit_v
                packed = (gpos << 17) | (fl & (RANGE - 1))
                pc = plsc.cumsum(m.astype(jnp.int32))
                off = cnt_v + pc - 1
                plsc.store_scatter(pk, [off], packed, mask=m)
                cnt_v = cnt_v + plsc.all_reduce_population_count(m)
                return cnt_v, gpos + L
            cnt_v, _ = lax.fori_loop(0, SGRP, scan_body,
                                     (zeros_i, seg * SEG + iota), unroll=4)
            cnt = jnp.max(cnt_v)

            nblk = (cnt + BLK - 1) // BLK

            def blk_body(blk, hw):
                cb = jnp.minimum(cnt - blk * BLK, BLK)
                ng = (cb + 127) >> 7
                cb_v = jnp.full((L,), cb, jnp.int32)

                def ext_body(q, _):
                    q16 = iota + q * L
                    sel = q16 < cb_v
                    pkv = pk[pl.ds(blk * BLK + q * L, L)]
                    pos = lax.shift_right_logical(pkv, 17)
                    lv = pkv & (RANGE - 1)
                    plsc.store_scatter(lvb,
                                       [jnp.full((L,), q >> 3, jnp.int32),
                                        zeros_i, (q & 7) * L + iota], lv)
                    for k in range(4):
                        g32 = plsc.load_gather(vres, [pos + k * TPT])
                        bf = plsc.bitcast(g32, jnp.bfloat16)
                        a, b2 = plsc.unpack(
                            bf, format=plsc.PackFormat.INTERLEAVED)
                        pays[2 * k][pl.ds(q * L, L)] = jnp.where(sel, a, 0.0)
                        pays[2 * k + 1][pl.ds(q * L, L)] = jnp.where(
                            sel, b2, 0.0)
                    return _
                nq = (cb + L - 1) >> 4
                lax.fori_loop(0, nq, ext_body, None)

                def zpay(r, _):
                    for b in range(BATCH):
                        pays[b][pl.ds((nq + r) * L, L)] = zeros_f
                    return _
                lax.fori_loop(0, jnp.maximum(hw - nq, 0), zpay, None)
                hw = jnp.maximum(hw, nq)

                def st_body(j, _):
                    for b in range(BATCH):
                        pltpu.async_copy(pays[b].at[pl.ds(j * 128, 128)],
                                         acc[b].at[lvb.at[j, 0]], ssem,
                                         add=True)
                    for b in range(BATCH):
                        pltpu.make_async_copy(
                            pays[b].at[pl.ds(j * 128, 128)],
                            acc[b].at[lvb.at[j, 0]], ssem).wait()
                    return _
                lax.fori_loop(0, ng, st_body, None)
                return hw
            return lax.fori_loop(0, nblk, blk_body, hw)
        hw = lax.fori_loop(0, NSEG, seg_body, hw)

        plsc.subcore_barrier()
        for b in range(BATCH):
            pltpu.async_copy(
                acc[b].at[pl.ds(s * SLICE, SLICE)],
                out_hbm.at[pl.ds(b * NVOX + base + s * SLICE, SLICE)], rsem)
        for b in range(BATCH):
            pltpu.make_async_copy(
                acc[b].at[pl.ds(s * SLICE, SLICE)],
                out_hbm.at[pl.ds(b * NVOX + base + s * SLICE, SLICE)],
                rsem).wait()
        return hw

    lax.fori_loop(0, NPASS, pass_body, jnp.int32(0))


def _scatter(flat, values_p):
    mesh = plsc.VectorSubcoreMesh(core_axis_name="c", subcore_axis_name="s",
                                  num_cores=NC, num_subcores=NS)
    return pl.kernel(
        _scatter_body,
        out_type=jax.ShapeDtypeStruct((BATCH * NVOX,), jnp.float32),
        mesh=mesh,
        scratch_types=[
            pltpu.VMEM((TPT,), jnp.int32),
            pltpu.VMEM((TPT * 4,), jnp.int32),
            pltpu.VMEM((CAP,), jnp.int32),
            pltpu.VMEM((NCH, 1, 128), jnp.int32),
            [pltpu.VMEM((BLK,), jnp.float32) for _ in range(BATCH)],
            pltpu.VMEM((ZB,), jnp.float32),
            [pltpu.VMEM_SHARED((RANGE,), jnp.float32)
             for _ in range(BATCH)],
            pltpu.SemaphoreType.DMA,
            pltpu.SemaphoreType.DMA,
            pltpu.SemaphoreType.DMA,
        ],
        compiler_params=pltpu.CompilerParams(needs_layout_passes=False),
    )(flat, values_p)


def kernel(x, inds, reference_values, W0, b0, W1, b1, W2, b2, W3, b3, W4, b4,
           W5, b5):
    values_p, flat = _values_and_flat(
        x, inds, reference_values, W0, b0, W1, b1, W2, b2, W3, b3, W4, b4,
        W5, b5)
    grids = _scatter(flat, values_p)
    return grids.reshape(BATCH, VOLUME_SIZE, VOLUME_SIZE, VOLUME_SIZE)


# speedup vs baseline: 1.1879x; 1.0568x over previous
import jax
import jax.numpy as jnp
from jax import lax
from jax.experimental import pallas as pl
from jax.experimental.pallas import tpu as pltpu
from jax.experimental.pallas import tpu_sc as plsc

T = 262144
VOLUME_SIZE = 128
NVOX = VOLUME_SIZE ** 3
BATCH = 8


TBLK = 32768
NTBLK = T // TBLK


def _rne_bf16_bits(u):
    return (u + 0x7FFF + ((u >> 16) & 1)) & jnp.uint32(0xFFFF0000)


def _values_body(x_ref, w0_ref, b0_ref, w1_ref, b1_ref, w2_ref, b2_ref,
                 w3_ref, b3_ref, w4_ref, b4_ref, w5_ref, b5_ref, rv_ref,
                 i0_ref, i1_ref, i2_ref, valp_ref, flat_ref):
    h = jnp.sin(30.0 * (jnp.dot(x_ref[...], w0_ref[...],
                                preferred_element_type=jnp.float32)
                        + b0_ref[...]))
    for w_ref, b_ref in ((w1_ref, b1_ref), (w2_ref, b2_ref),
                         (w3_ref, b3_ref), (w4_ref, b4_ref)):
        h = h + jnp.sin(jnp.dot(h, w_ref[...],
                                preferred_element_type=jnp.float32)
                        + b_ref[...])
    out = jnp.dot(h, w5_ref[...], preferred_element_type=jnp.float32)
    out = out + b5_ref[0] + rv_ref[0]
    bits = _rne_bf16_bits(lax.bitcast_convert_type(out, jnp.uint32))
    rows = []
    for k in range(4):
        pair = (bits[2 * k:2 * k + 1] >> 16) | bits[2 * k + 1:2 * k + 2]
        rows.append(lax.bitcast_convert_type(pair, jnp.int32).reshape(TBLK))
    valp_ref[...] = jnp.concatenate(rows)
    flat_ref[...] = (i0_ref[...] * (VOLUME_SIZE * VOLUME_SIZE)
                     + i1_ref[...] * VOLUME_SIZE
                     + i2_ref[...]).reshape(TBLK)


def _values_and_flat(x, inds, reference_values, W0, b0, W1, b1, W2, b2,
                     W3, b3, W4, b4, W5, b5):
    i0 = inds[:, 0].reshape(NTBLK, 1, TBLK)
    i1 = inds[:, 1].reshape(NTBLK, 1, TBLK)
    i2 = inds[:, 2].reshape(NTBLK, 1, TBLK)
    b5r = b5.reshape(NTBLK, 1, TBLK)
    rvr = reference_values.reshape(NTBLK, 1, TBLK)

    def full(shape):
        return pl.BlockSpec(shape, lambda i: (0,) * len(shape))

    tb3 = pl.BlockSpec((1, 1, TBLK), lambda i: (i, 0, 0))
    valp, flat = pl.pallas_call(
        _values_body,
        grid=(NTBLK,),
        in_specs=[
            full((BATCH, 64)), full((64, 8)), full((1, 8)),
            full((8, 8)), full((1, 8)), full((8, 8)), full((1, 8)),
            full((8, 8)), full((1, 8)), full((8, 8)), full((1, 8)),
            pl.BlockSpec((BATCH, TBLK), lambda i: (0, i)),
            tb3, tb3,
            tb3, tb3, tb3,
        ],
        out_specs=[
            pl.BlockSpec((TBLK * 4,), lambda i: (i,)),
            pl.BlockSpec((TBLK,), lambda i: (i,)),
        ],
        out_shape=[
            jax.ShapeDtypeStruct((T * 4,), jnp.int32),
            jax.ShapeDtypeStruct((T,), jnp.int32),
        ],
    )(x, W0, b0.reshape(1, -1), W1, b1.reshape(1, -1), W2, b2.reshape(1, -1),
      W3, b3.reshape(1, -1), W4, b4.reshape(1, -1), W5, b5r, rvr, i0, i1, i2)
    return valp, flat


NC = 2
NS = 16
L = 16
NPASS = 16
RANGE = NVOX // (NPASS * NC)
RSHIFT = 16
SLICE = RANGE // NS
ZB = 1024
TPT = T // NS
NGRP = TPT // L
BLK = 1024
NCH = BLK // 128
NQ = BLK // L
SEG = 4096
NSEG = TPT // SEG
SGRP = SEG // L
CAP = SEG + 128


def _scatter_body(flat_hbm, vp_hbm, out_hbm,
                  idx_res, vres, pk, lvb, pays, zbuf,
                  acc, ssem, zsem, rsem):
    c = lax.axis_index("c")
    s = lax.axis_index("s")
    tb = s * TPT
    iota = lax.iota(jnp.int32, L)
    zeros_i = jnp.zeros((L,), jnp.int32)
    zeros_f = jnp.zeros((L,), jnp.float32)

    pltpu.sync_copy(flat_hbm.at[pl.ds(tb, TPT)], idx_res)
    vpb = (s // 2) * (TBLK * 4) + (s % 2) * TPT
    for k in range(4):
        pltpu.sync_copy(vp_hbm.at[pl.ds(vpb + k * TBLK, TPT)],
                        vres.at[pl.ds(k * TPT, TPT)])

    def zinit(j, _):
        zbuf[pl.ds(j * L, L)] = zeros_f
        return _
    lax.fori_loop(0, ZB // L, zinit, None, unroll=8)

    def pinit(j, _):
        for b in range(BATCH):
            pays[b][pl.ds(j * L, L)] = zeros_f
        return _
    lax.fori_loop(0, NQ, pinit, None, unroll=4)

    def cinit(j, _):
        pk[pl.ds(j * L, L)] = zeros_i
        return _
    lax.fori_loop(0, CAP // L, cinit, None, unroll=4)

    def linit(j, _):
        off = j * L + iota
        plsc.store_scatter(lvb, [off >> 7, zeros_i, off & 127], iota * 8)
        return _
    lax.fori_loop(0, BLK // L, linit, None, unroll=4)

    def issue_zero():
        for b in range(BATCH):
            for h in range(SLICE // ZB):
                pltpu.async_copy(
                    zbuf, acc[b].at[pl.ds(s * SLICE + h * ZB, ZB)], zsem)

    def drain_zero():
        for b in range(BATCH):
            for h in range(SLICE // ZB):
                pltpu.make_async_copy(
                    zbuf, acc[b].at[pl.ds(s * SLICE + h * ZB, ZB)],
                    zsem).wait()

    issue_zero()

    def pass_body(p, hw):
        unit = p * NC + c
        base = unit * RANGE
        unit_v = jnp.full((L,), unit, jnp.int32)

        def scan_seg(seg):
            def scan_body(g, carry):
                cnt_v, gpos = carry
                fl = idx_res[pl.ds(seg * SEG + g * L, L)]
                m = (fl >> RSHIFT) == unit_v
                packed = (gpos << 17) | (fl & (RANGE - 1))
                pc = plsc.cumsum(m.astype(jnp.int32))
                off = cnt_v + pc - 1
                plsc.store_scatter(pk, [off], packed, mask=m)
                cnt_v = cnt_v + plsc.all_reduce_population_count(m)
                return cnt_v, gpos + L
            cnt_v, _ = lax.fori_loop(0, SGRP, scan_body,
                                     (zeros_i, seg * SEG + iota), unroll=4)
            return jnp.max(cnt_v)

        def process_seg(cnt, hw):
            nblk = (cnt + BLK - 1) // BLK

            def blk_body(blk, hw):
                cb = jnp.minimum(cnt - blk * BLK, BLK)
                ng = (cb + 127) >> 7
                cb_v = jnp.full((L,), cb, jnp.int32)

                def ext_body(q, _):
                    q16 = iota + q * L
                    sel = q16 < cb_v
                    pkv = pk[pl.ds(blk * BLK + q * L, L)]
                    pos = lax.shift_right_logical(pkv, 17)
                    lv = pkv & (RANGE - 1)
                    plsc.store_scatter(lvb,
                                       [jnp.full((L,), q >> 3, jnp.int32),
                                        zeros_i, (q & 7) * L + iota], lv)
                    for k in range(4):
                        g32 = plsc.load_gather(vres, [pos + k * TPT])
                        bf = plsc.bitcast(g32, jnp.bfloat16)
                        a, b2 = plsc.unpack(
                            bf, format=plsc.PackFormat.INTERLEAVED)
                        pays[2 * k][pl.ds(q * L, L)] = jnp.where(sel, a, 0.0)
                        pays[2 * k + 1][pl.ds(q * L, L)] = jnp.where(
                            sel, b2, 0.0)
                    return _
                nq = (cb + L - 1) >> 4
                lax.fori_loop(0, nq, ext_body, None)

                def zpay(r, _):
                    for b in range(BATCH):
                        pays[b][pl.ds((nq + r) * L, L)] = zeros_f
                    return _
                lax.fori_loop(0, jnp.maximum(hw - nq, 0), zpay, None)
                hw = jnp.maximum(hw, nq)

                def st_issue(j, _):
                    for b in range(BATCH):
                        pltpu.async_copy(pays[b].at[pl.ds(j * 128, 128)],
                                         acc[b].at[lvb.at[j, 0]], ssem,
                                         add=True)
                    return _
                lax.fori_loop(0, ng, st_issue, None)

                def st_drain(j, _):
                    for b in range(BATCH):
                        pltpu.make_async_copy(
                            pays[b].at[pl.ds(j * 128, 128)],
                            acc[b].at[lvb.at[j, 0]], ssem).wait()
                    return _
                lax.fori_loop(0, ng, st_drain, None)
                return hw
            return lax.fori_loop(0, nblk, blk_body, hw)

        cnt0 = scan_seg(0)
        drain_zero()
        plsc.subcore_barrier()
        hw = process_seg(cnt0, hw)

        def seg_body(seg, hw):
            return process_seg(scan_seg(seg), hw)
        hw = lax.fori_loop(1, NSEG, seg_body, hw)

        plsc.subcore_barrier()
        for b in range(BATCH):
            pltpu.async_copy(
                acc[b].at[pl.ds(s * SLICE, SLICE)],
                out_hbm.at[pl.ds(b * NVOX + base + s * SLICE, SLICE)], rsem)
        for b in range(BATCH):
            pltpu.make_async_copy(
                acc[b].at[pl.ds(s * SLICE, SLICE)],
                out_hbm.at[pl.ds(b * NVOX + base + s * SLICE, SLICE)],
                rsem).wait()
        issue_zero()
        return hw

    lax.fori_loop(0, NPASS, pass_body, jnp.int32(0))
    drain_zero()


def _scatter(flat, values_p):
    mesh = plsc.VectorSubcoreMesh(core_axis_name="c", subcore_axis_name="s",
                                  num_cores=NC, num_subcores=NS)
    return pl.kernel(
        _scatter_body,
        out_type=jax.ShapeDtypeStruct((BATCH * NVOX,), jnp.float32),
        mesh=mesh,
        scratch_types=[
            pltpu.VMEM((TPT,), jnp.int32),
            pltpu.VMEM((TPT * 4,), jnp.int32),
            pltpu.VMEM((CAP,), jnp.int32),
            pltpu.VMEM((NCH, 1, 128), jnp.int32),
            [pltpu.VMEM((BLK,), jnp.float32) for _ in range(BATCH)],
            pltpu.VMEM((ZB,), jnp.float32),
            [pltpu.VMEM_SHARED((RANGE,), jnp.float32)
             for _ in range(BATCH)],
            pltpu.SemaphoreType.DMA,
            pltpu.SemaphoreType.DMA,
            pltpu.SemaphoreType.DMA,
        ],
        compiler_params=pltpu.CompilerParams(needs_layout_passes=False),
    )(flat, values_p)


def kernel(x, inds, reference_values, W0, b0, W1, b1, W2, b2, W3, b3, W4, b4,
           W5, b5):
    values_p, flat = _values_and_flat(
        x, inds, reference_values, W0, b0, W1, b1, W2, b2, W3, b3, W4, b4,
        W5, b5)
    grids = _scatter(flat, values_p)
    return grids.reshape(BATCH, VOLUME_SIZE, VOLUME_SIZE, VOLUME_SIZE)


# speedup vs baseline: 1.2177x; 1.0251x over previous
import jax
import jax.numpy as jnp
from jax import lax
from jax.experimental import pallas as pl
from jax.experimental.pallas import tpu as pltpu
from jax.experimental.pallas import tpu_sc as plsc

T = 262144
VOLUME_SIZE = 128
NVOX = VOLUME_SIZE ** 3
BATCH = 8


TBLK = 32768
NTBLK = T // TBLK


def _rne_bf16_bits(u):
    return (u + 0x7FFF + ((u >> 16) & 1)) & jnp.uint32(0xFFFF0000)


def _values_body(x_ref, w0_ref, b0_ref, w1_ref, b1_ref, w2_ref, b2_ref,
                 w3_ref, b3_ref, w4_ref, b4_ref, w5_ref, b5_ref, rv_ref,
                 i0_ref, i1_ref, i2_ref, valp_ref, flat_ref):
    h = jnp.sin(30.0 * (jnp.dot(x_ref[...], w0_ref[...],
                                preferred_element_type=jnp.float32)
                        + b0_ref[...]))
    for w_ref, b_ref in ((w1_ref, b1_ref), (w2_ref, b2_ref),
                         (w3_ref, b3_ref), (w4_ref, b4_ref)):
        h = h + jnp.sin(jnp.dot(h, w_ref[...],
                                preferred_element_type=jnp.float32)
                        + b_ref[...])
    out = jnp.dot(h, w5_ref[...], preferred_element_type=jnp.float32)
    out = out + b5_ref[0] + rv_ref[0]
    bits = _rne_bf16_bits(lax.bitcast_convert_type(out, jnp.uint32))
    rows = []
    for k in range(4):
        pair = (bits[2 * k:2 * k + 1] >> 16) | bits[2 * k + 1:2 * k + 2]
        rows.append(lax.bitcast_convert_type(pair, jnp.int32).reshape(TBLK))
    valp_ref[...] = jnp.concatenate(rows)
    flat_ref[...] = (i0_ref[...] * (VOLUME_SIZE * VOLUME_SIZE)
                     + i1_ref[...] * VOLUME_SIZE
                     + i2_ref[...]).reshape(TBLK)


def _values_and_flat(x, inds, reference_values, W0, b0, W1, b1, W2, b2,
                     W3, b3, W4, b4, W5, b5):
    i0 = inds[:, 0].reshape(NTBLK, 1, TBLK)
    i1 = inds[:, 1].reshape(NTBLK, 1, TBLK)
    i2 = inds[:, 2].reshape(NTBLK, 1, TBLK)
    b5r = b5.reshape(NTBLK, 1, TBLK)
    rvr = reference_values.reshape(NTBLK, 1, TBLK)

    def full(shape):
        return pl.BlockSpec(shape, lambda i: (0,) * len(shape))

    tb3 = pl.BlockSpec((1, 1, TBLK), lambda i: (i, 0, 0))
    valp, flat = pl.pallas_call(
        _values_body,
        grid=(NTBLK,),
        in_specs=[
            full((BATCH, 64)), full((64, 8)), full((1, 8)),
            full((8, 8)), full((1, 8)), full((8, 8)), full((1, 8)),
            full((8, 8)), full((1, 8)), full((8, 8)), full((1, 8)),
            pl.BlockSpec((BATCH, TBLK), lambda i: (0, i)),
            tb3, tb3,
            tb3, tb3, tb3,
        ],
        out_specs=[
            pl.BlockSpec((TBLK * 4,), lambda i: (i,)),
            pl.BlockSpec((TBLK,), lambda i: (i,)),
        ],
        out_shape=[
            jax.ShapeDtypeStruct((T * 4,), jnp.int32),
            jax.ShapeDtypeStruct((T,), jnp.int32),
        ],
    )(x, W0, b0.reshape(1, -1), W1, b1.reshape(1, -1), W2, b2.reshape(1, -1),
      W3, b3.reshape(1, -1), W4, b4.reshape(1, -1), W5, b5r, rvr, i0, i1, i2)
    return valp, flat


NC = 2
NS = 16
L = 16
NPASS = 16
RANGE = NVOX // (NPASS * NC)
RSHIFT = 16
SLICE = RANGE // NS
ZB = 1024
TPT = T // NS
NGRP = TPT // L
BLK = 1024
NCH = BLK // 128
NQ = BLK // L
SEG = 4096
NSEG = TPT // SEG
SGRP = SEG // L
CAP = SEG + 128


def _scatter_body(flat_hbm, vp_hbm, out_hbm,
                  idx_res, vres, pk, lvb, pays, zbuf,
                  acc, ssem, zsem, rsem):
    c = lax.axis_index("c")
    s = lax.axis_index("s")
    tb = s * TPT
    iota = lax.iota(jnp.int32, L)
    zeros_i = jnp.zeros((L,), jnp.int32)
    zeros_f = jnp.zeros((L,), jnp.float32)

    pltpu.sync_copy(flat_hbm.at[pl.ds(tb, TPT)], idx_res)
    vpb = (s // 2) * (TBLK * 4) + (s % 2) * TPT
    for k in range(4):
        pltpu.sync_copy(vp_hbm.at[pl.ds(vpb + k * TBLK, TPT)],
                        vres.at[pl.ds(k * TPT, TPT)])

    def zinit(j, _):
        zbuf[pl.ds(j * L, L)] = zeros_f
        return _
    lax.fori_loop(0, ZB // L, zinit, None, unroll=8)

    def pinit(j, _):
        for b in range(BATCH):
            pays[b][pl.ds(j * L, L)] = zeros_f
        return _
    lax.fori_loop(0, NQ, pinit, None, unroll=4)

    def cinit(j, _):
        pk[pl.ds(j * L, L)] = zeros_i
        return _
    lax.fori_loop(0, CAP // L, cinit, None, unroll=4)

    def linit(j, _):
        off = j * L + iota
        plsc.store_scatter(lvb, [off >> 7, zeros_i, off & 127], iota * 8)
        return _
    lax.fori_loop(0, BLK // L, linit, None, unroll=4)

    def issue_zero():
        for b in range(BATCH):
            for h in range(SLICE // ZB):
                pltpu.async_copy(
                    zbuf, acc[b].at[pl.ds(s * SLICE + h * ZB, ZB)], zsem)

    def drain_zero():
        for b in range(BATCH):
            for h in range(SLICE // ZB):
                pltpu.make_async_copy(
                    zbuf, acc[b].at[pl.ds(s * SLICE + h * ZB, ZB)],
                    zsem).wait()

    issue_zero()

    def pass_body(p, hw):
        unit = p * NC + c
        base = unit * RANGE
        unit_v = jnp.full((L,), unit, jnp.int32)

        base_v = unit_v * RANGE

        def scan_seg(seg):
            def scan_body(g, carry):
                cnt1_v, gpos_sh = carry
                fl = idx_res[pl.ds(seg * SEG + g * L, L)]
                lvu = fl - base_v
                m = (lax.bitcast_convert_type(lvu, jnp.uint32)
                     < jnp.uint32(RANGE))
                packed = gpos_sh | lvu
                pc = plsc.cumsum(m.astype(jnp.int32))
                off = cnt1_v + pc
                plsc.store_scatter(pk, [off], packed, mask=m)
                cnt1_v = cnt1_v + plsc.all_reduce_population_count(m)
                return cnt1_v, gpos_sh + (L << 17)
            cnt1_v, _ = lax.fori_loop(
                0, SGRP, scan_body,
                (zeros_i - 1, (seg * SEG + iota) << 17), unroll=4)
            return jnp.max(cnt1_v) + 1

        def process_seg(cnt, hw):
            nblk = (cnt + BLK - 1) // BLK

            def blk_body(blk, hw):
                cb = jnp.minimum(cnt - blk * BLK, BLK)
                ng = (cb + 127) >> 7
                cb_v = jnp.full((L,), cb, jnp.int32)

                def ext_body(q, _):
                    q16 = iota + q * L
                    sel = q16 < cb_v
                    pkv = pk[pl.ds(blk * BLK + q * L, L)]
                    pos = lax.shift_right_logical(pkv, 17)
                    lv = pkv & (RANGE - 1)
                    plsc.store_scatter(lvb,
                                       [jnp.full((L,), q >> 3, jnp.int32),
                                        zeros_i, (q & 7) * L + iota], lv)
                    for k in range(4):
                        g32 = plsc.load_gather(vres, [pos + k * TPT])
                        bf = plsc.bitcast(g32, jnp.bfloat16)
                        a, b2 = plsc.unpack(
                            bf, format=plsc.PackFormat.INTERLEAVED)
                        pays[2 * k][pl.ds(q * L, L)] = jnp.where(sel, a, 0.0)
                        pays[2 * k + 1][pl.ds(q * L, L)] = jnp.where(
                            sel, b2, 0.0)
                    return _
                nq = (cb + L - 1) >> 4
                lax.fori_loop(0, nq, ext_body, None)

                def zpay(r, _):
                    for b in range(BATCH):
                        pays[b][pl.ds((nq + r) * L, L)] = zeros_f
                    return _
                lax.fori_loop(0, jnp.maximum(hw - nq, 0), zpay, None)
                hw = jnp.maximum(hw, nq)

                def st_issue(j, _):
                    for b in range(BATCH):
                        pltpu.async_copy(pays[b].at[pl.ds(j * 128, 128)],
                                         acc[b].at[lvb.at[j, 0]], ssem,
                                         add=True)
                    return _
                lax.fori_loop(0, ng, st_issue, None)

                def st_drain(j, _):
                    for b in range(BATCH):
                        pltpu.make_async_copy(
                            pays[b].at[pl.ds(j * 128, 128)],
                            acc[b].at[lvb.at[j, 0]], ssem).wait()
                    return _
                lax.fori_loop(0, ng, st_drain, None)
                return hw
            return lax.fori_loop(0, nblk, blk_body, hw)

        cnt0 = scan_seg(0)
        drain_zero()
        plsc.subcore_barrier()
        hw = process_seg(cnt0, hw)

        def seg_body(seg, hw):
            return process_seg(scan_seg(seg), hw)
        hw = lax.fori_loop(1, NSEG, seg_body, hw)

        plsc.subcore_barrier()
        for b in range(BATCH):
            pltpu.async_copy(
                acc[b].at[pl.ds(s * SLICE, SLICE)],
                out_hbm.at[pl.ds(b * NVOX + base + s * SLICE, SLICE)], rsem)
        for b in range(BATCH):
            pltpu.make_async_copy(
                acc[b].at[pl.ds(s * SLICE, SLICE)],
                out_hbm.at[pl.ds(b * NVOX + base + s * SLICE, SLICE)],
                rsem).wait()
        issue_zero()
        return hw

    lax.fori_loop(0, NPASS, pass_body, jnp.int32(0))
    drain_zero()


def _scatter(flat, values_p):
    mesh = plsc.VectorSubcoreMesh(core_axis_name="c", subcore_axis_name="s",
                                  num_cores=NC, num_subcores=NS)
    return pl.kernel(
        _scatter_body,
        out_type=jax.ShapeDtypeStruct((BATCH * NVOX,), jnp.float32),
        mesh=mesh,
        scratch_types=[
            pltpu.VMEM((TPT,), jnp.int32),
            pltpu.VMEM((TPT * 4,), jnp.int32),
            pltpu.VMEM((CAP,), jnp.int32),
            pltpu.VMEM((NCH, 1, 128), jnp.int32),
            [pltpu.VMEM((BLK,), jnp.float32) for _ in range(BATCH)],
            pltpu.VMEM((ZB,), jnp.float32),
            [pltpu.VMEM_SHARED((RANGE,), jnp.float32)
             for _ in range(BATCH)],
            pltpu.SemaphoreType.DMA,
            pltpu.SemaphoreType.DMA,
            pltpu.SemaphoreType.DMA,
        ],
        compiler_params=pltpu.CompilerParams(needs_layout_passes=False),
    )(flat, values_p)


def kernel(x, inds, reference_values, W0, b0, W1, b1, W2, b2, W3, b3, W4, b4,
           W5, b5):
    values_p, flat = _values_and_flat(
        x, inds, reference_values, W0, b0, W1, b1, W2, b2, W3, b3, W4, b4,
        W5, b5)
    grids = _scatter(flat, values_p)
    return grids.reshape(BATCH, VOLUME_SIZE, VOLUME_SIZE, VOLUME_SIZE)
